# baseline (speedup 1.0000x reference)
import jax
import jax.numpy as jnp
from jax.experimental import pallas as pl
from jax.experimental.pallas import tpu as pltpu


def _pick_bm(half):
    for bm in (200, 100, 40, 8):
        if half % bm == 0 and bm % 8 == 0:
            return bm
    return half


def _h_kernel(x_ref, w0_ref, adj_a, adj_b, h_a, h_b, xw0_ref):
    @pl.when(pl.program_id(0) == 0)
    def _():
        xw0_ref[...] = jnp.dot(
            x_ref[...], w0_ref[...], preferred_element_type=jnp.float32
        ).astype(jnp.bfloat16)

    xw0 = xw0_ref[...]
    h_a[...] = jnp.tanh(
        jnp.dot(
            adj_a[...].astype(jnp.bfloat16), xw0, preferred_element_type=jnp.float32
        )
    )
    h_b[...] = jnp.tanh(
        jnp.dot(
            adj_b[...].astype(jnp.bfloat16), xw0, preferred_element_type=jnp.float32
        )
    )


def _head_half(adj_ref, eps_ref, hw, latent, z_ref, m_ref, s_ref):
    acc = jnp.dot(
        adj_ref[...].astype(jnp.bfloat16), hw, preferred_element_type=jnp.float32
    )
    m = acc[:, :latent]
    s = jnp.maximum(acc[:, latent:], 0.0) + 0.0001
    m_ref[...] = m
    s_ref[...] = s
    z_ref[...] = eps_ref[...] * s + m


def _head_kernel(
    h_ref, wcat_ref, adj_a, adj_b, eps_a, eps_b,
    z_a, m_a, s_a, z_b, m_b, s_b, hw_ref,
):
    latent = m_a.shape[1]

    @pl.when(pl.program_id(0) == 0)
    def _():
        hw_ref[...] = jnp.dot(
            h_ref[...], wcat_ref[...], preferred_element_type=jnp.float32
        ).astype(jnp.bfloat16)

    hw = hw_ref[...]
    _head_half(adj_a, eps_a, hw, latent, z_a, m_a, s_a)
    _head_half(adj_b, eps_b, hw, latent, z_b, m_b, s_b)


def kernel(adj, x, W0, Wm, Ws):
    n, d_in = x.shape
    hidden = W0.shape[1]
    latent = Wm.shape[1]
    half = n // 2
    bm = _pick_bm(half)
    nb = half // bm
    grid = (nb,)

    def row_spec(off):
        return pl.BlockSpec((bm, n), lambda i, off=off: (i + off, 0))

    h_sds = jax.ShapeDtypeStruct((half, hidden), jnp.float32)
    h_a, h_b = pl.pallas_call(
        _h_kernel,
        grid=grid,
        in_specs=[
            pl.BlockSpec((n, d_in), lambda i: (0, 0)),
            pl.BlockSpec((d_in, hidden), lambda i: (0, 0)),
            row_spec(0),
            row_spec(nb),
        ],
        out_specs=[
            pl.BlockSpec((bm, hidden), lambda i: (i, 0)),
            pl.BlockSpec((bm, hidden), lambda i: (i, 0)),
        ],
        out_shape=[h_sds, h_sds],
        scratch_shapes=[pltpu.VMEM((n, hidden), jnp.bfloat16)],
        compiler_params=pltpu.CompilerParams(
            dimension_semantics=("arbitrary",),
        ),
    )(x, W0, adj, adj)

    h = jnp.concatenate([h_a, h_b], axis=0)
    wcat = jnp.concatenate([Wm, Ws], axis=1)
    eps = jax.random.normal(jax.random.key(42), (n, latent), dtype=jnp.float32)

    def lat_spec():
        return pl.BlockSpec((bm, latent), lambda i: (i, 0))

    def eps_spec(off):
        return pl.BlockSpec((bm, latent), lambda i, off=off: (i + off, 0))

    out_sds = jax.ShapeDtypeStruct((half, latent), jnp.float32)
    z_a, m_a, s_a, z_b, m_b, s_b = pl.pallas_call(
        _head_kernel,
        grid=grid,
        in_specs=[
            pl.BlockSpec((n, hidden), lambda i: (0, 0)),
            pl.BlockSpec((hidden, 2 * latent), lambda i: (0, 0)),
            row_spec(0),
            row_spec(nb),
            eps_spec(0),
            eps_spec(nb),
        ],
        out_specs=[lat_spec() for _ in range(6)],
        out_shape=[out_sds] * 6,
        scratch_shapes=[pltpu.VMEM((n, 2 * latent), jnp.bfloat16)],
        compiler_params=pltpu.CompilerParams(
            dimension_semantics=("arbitrary",),
        ),
    )(h, wcat, adj, adj, eps, eps)

    z = jnp.concatenate([z_a, z_b], axis=0)
    m_q_z = jnp.concatenate([m_a, m_b], axis=0)
    std_q_z = jnp.concatenate([s_a, s_b], axis=0)
    return (z, m_q_z, std_q_z)


# speedup vs baseline: 1.0368x; 1.0368x over previous
import jax
import jax.numpy as jnp
from jax.experimental import pallas as pl
from jax.experimental.pallas import tpu as pltpu


def _block_sizes(n):
    for bm in (400, 200, 80, 16, 8):
        if n % bm == 0:
            for bmc in (40, 16, 8):
                if bm % bmc == 0:
                    return bm, bmc
    return n, n


def _chunk_copies(adj_hbm, buf_ref, sems, blk, slot, bm, bmc):
    n_chunks = bm // bmc
    return [
        pltpu.make_async_copy(
            adj_hbm.at[pl.ds(blk * bm + c * bmc, bmc), :],
            buf_ref.at[slot, pl.ds(c * bmc, bmc), :],
            sems.at[slot, c],
        )
        for c in range(n_chunks)
    ]


def _pipeline_block(adj_hbm, buf_ref, sems, nb, bm, bmc):
    i = pl.program_id(0)
    slot = jax.lax.rem(i, 2)
    nxt = jax.lax.rem(i + 1, 2)

    @pl.when(i == 0)
    def _():
        for cp in _chunk_copies(adj_hbm, buf_ref, sems, i, slot, bm, bmc):
            cp.start()

    @pl.when(i + 1 < nb)
    def _():
        for cp in _chunk_copies(adj_hbm, buf_ref, sems, i + 1, nxt, bm, bmc):
            cp.start()

    for cp in _chunk_copies(adj_hbm, buf_ref, sems, i, slot, bm, bmc):
        cp.wait()
    return buf_ref[slot]


def _h_kernel(nb, bm, bmc, x_ref, w0_ref, adj_hbm, h_ref, xw0_ref, buf_ref, sems):
    @pl.when(pl.program_id(0) == 0)
    def _():
        xw0_ref[...] = jnp.dot(
            x_ref[...], w0_ref[...], preferred_element_type=jnp.float32
        ).astype(jnp.bfloat16)

    adj_blk = _pipeline_block(adj_hbm, buf_ref, sems, nb, bm, bmc)
    h_ref[...] = jnp.tanh(
        jnp.dot(
            adj_blk.astype(jnp.bfloat16),
            xw0_ref[...],
            preferred_element_type=jnp.float32,
        )
    )


def _head_kernel(
    nb, bm, bmc, h_ref, wcat_ref, adj_hbm, eps_ref,
    z_ref, m_ref, s_ref, hw_ref, buf_ref, sems,
):
    latent = m_ref.shape[1]

    @pl.when(pl.program_id(0) == 0)
    def _():
        hw_ref[...] = jnp.dot(
            h_ref[...], wcat_ref[...], preferred_element_type=jnp.float32
        ).astype(jnp.bfloat16)

    adj_blk = _pipeline_block(adj_hbm, buf_ref, sems, nb, bm, bmc)
    acc = jnp.dot(
        adj_blk.astype(jnp.bfloat16), hw_ref[...], preferred_element_type=jnp.float32
    )
    m = acc[:, :latent]
    s = jnp.maximum(acc[:, latent:], 0.0) + 0.0001
    m_ref[...] = m
    s_ref[...] = s
    z_ref[...] = eps_ref[...] * s + m


def kernel(adj, x, W0, Wm, Ws):
    import functools

    n, d_in = x.shape
    hidden = W0.shape[1]
    latent = Wm.shape[1]
    bm, bmc = _block_sizes(n)
    nb = n // bm
    n_chunks = bm // bmc
    grid = (nb,)

    adj_spec = pl.BlockSpec(memory_space=pl.ANY)
    dma_scratch = [
        pltpu.VMEM((2, bm, n), jnp.float32),
        pltpu.SemaphoreType.DMA((2, n_chunks)),
    ]

    h = pl.pallas_call(
        functools.partial(_h_kernel, nb, bm, bmc),
        grid=grid,
        in_specs=[
            pl.BlockSpec((n, d_in), lambda i: (0, 0)),
            pl.BlockSpec((d_in, hidden), lambda i: (0, 0)),
            adj_spec,
        ],
        out_specs=pl.BlockSpec((bm, hidden), lambda i: (i, 0)),
        out_shape=jax.ShapeDtypeStruct((n, hidden), jnp.float32),
        scratch_shapes=[pltpu.VMEM((n, hidden), jnp.bfloat16)] + dma_scratch,
        compiler_params=pltpu.CompilerParams(
            dimension_semantics=("arbitrary",),
        ),
    )(x, W0, adj)

    wcat = jnp.concatenate([Wm, Ws], axis=1)
    eps = jax.random.normal(jax.random.key(42), (n, latent), dtype=jnp.float32)

    out_sds = jax.ShapeDtypeStruct((n, latent), jnp.float32)
    lat_spec = pl.BlockSpec((bm, latent), lambda i: (i, 0))
    z, m_q_z, std_q_z = pl.pallas_call(
        functools.partial(_head_kernel, nb, bm, bmc),
        grid=grid,
        in_specs=[
            pl.BlockSpec((n, hidden), lambda i: (0, 0)),
            pl.BlockSpec((hidden, 2 * latent), lambda i: (0, 0)),
            adj_spec,
            lat_spec,
        ],
        out_specs=[lat_spec, lat_spec, lat_spec],
        out_shape=[out_sds, out_sds, out_sds],
        scratch_shapes=[pltpu.VMEM((n, 2 * latent), jnp.bfloat16)] + dma_scratch,
        compiler_params=pltpu.CompilerParams(
            dimension_semantics=("arbitrary",),
        ),
    )(h, wcat, adj, eps)

    return (z, m_q_z, std_q_z)
